# Initial kernel scaffold; baseline (speedup 1.0000x reference)
import jax
import jax.numpy as jnp
from jax.experimental import pallas as pl


def kernel(x, edge_index, W1, b1, g1, be1, W2, b2, g2, be2, W3, b3):
    raise NotImplementedError("write your pallas kernel here")


# speedup vs baseline: 9.8890x; 9.8890x over previous
import functools

import jax
import jax.numpy as jnp
from jax import lax
from jax.experimental import pallas as pl
from jax.experimental.pallas import tpu as pltpu
from jax.experimental.pallas import tpu_sc as plsc

N = 10000
NC, NS, L = 2, 16, 16
NW = NC * NS
K = 128
ROWS_PER_TILE = 640
N_ACC = NS * ROWS_PER_TILE
ZCHUNK = 16
EPS = 1e-5
SLOPE = 0.01


def _sc_mesh():
    return plsc.VectorSubcoreMesh(core_axis_name="c", subcore_axis_name="s")


def _make_deg_kernel(e_pad):
    epw = e_pad // NW
    n_chunks = epw // K

    @functools.partial(
        pl.kernel,
        out_type=jax.ShapeDtypeStruct((NC, N_ACC), jnp.float32),
        mesh=_sc_mesh(),
        scratch_types=[
            pltpu.VMEM((K,), jnp.int32),
            pltpu.VMEM((K,), jnp.float32),
            pltpu.VMEM((ROWS_PER_TILE,), jnp.float32),
            pltpu.VMEM_SHARED((N_ACC,), jnp.float32),
        ],
    )
    def deg_kernel(dst_hbm, out_hbm, dstv, ones_v, zv, acc):
        cid = lax.axis_index("c")
        sid = lax.axis_index("s")
        wid = cid * NS + sid

        def init_ones(i, _):
            ones_v[pl.ds(i * L, L)] = jnp.full((L,), 1.0, jnp.float32)
            return 0

        lax.fori_loop(0, K // L, init_ones, 0)

        def init_z(i, _):
            zv[pl.ds(i * L, L)] = jnp.zeros((L,), jnp.float32)
            return 0

        lax.fori_loop(0, ROWS_PER_TILE // L, init_z, 0)

        pltpu.sync_copy(zv, acc.at[pl.ds(sid * ROWS_PER_TILE, ROWS_PER_TILE)])
        plsc.subcore_barrier()

        def step(i, _):
            base = wid * epw + i * K
            pltpu.sync_copy(dst_hbm.at[pl.ds(base, K)], dstv)
            pltpu.sync_copy(ones_v, acc.at[dstv], add=True)
            return 0

        lax.fori_loop(0, n_chunks, step, 0)
        plsc.subcore_barrier()
        pltpu.sync_copy(
            acc.at[pl.ds(sid * ROWS_PER_TILE, ROWS_PER_TILE)],
            out_hbm.at[cid, pl.ds(sid * ROWS_PER_TILE, ROWS_PER_TILE)],
        )

    return deg_kernel


def _make_agg_kernel(e_pad, d):
    epw = e_pad // NW
    n_chunks = epw // K

    @functools.partial(
        pl.kernel,
        out_type=jax.ShapeDtypeStruct((NC, N_ACC, d), jnp.float32),
        mesh=_sc_mesh(),
        scratch_types=[
            pltpu.VMEM((K,), jnp.int32),
            pltpu.VMEM((K,), jnp.int32),
            pltpu.VMEM((K, d), jnp.float32),
            pltpu.VMEM((ZCHUNK, d), jnp.float32),
            pltpu.VMEM_SHARED((N_ACC, d), jnp.float32),
            pltpu.SemaphoreType.DMA,
        ],
    )
    def agg_kernel(hs_hbm, src_hbm, dst_hbm, out_hbm,
                   srcv, dstv, rows, zrows, acc, sem):
        cid = lax.axis_index("c")
        sid = lax.axis_index("s")
        wid = cid * NS + sid

        def zinit(i, _):
            r = i // (d // L)
            c = (i % (d // L)) * L
            zrows[r, pl.ds(c, L)] = jnp.zeros((L,), jnp.float32)
            return 0

        lax.fori_loop(0, ZCHUNK * d // L, zinit, 0)

        def zacc(i, _):
            pltpu.sync_copy(
                zrows, acc.at[pl.ds(sid * ROWS_PER_TILE + i * ZCHUNK, ZCHUNK)])
            return 0

        lax.fori_loop(0, ROWS_PER_TILE // ZCHUNK, zacc, 0)
        plsc.subcore_barrier()

        def step(i, _):
            base = wid * epw + i * K
            pltpu.sync_copy(src_hbm.at[pl.ds(base, K)], srcv)
            pltpu.sync_copy(dst_hbm.at[pl.ds(base, K)], dstv)
            pltpu.async_copy(hs_hbm.at[srcv], rows, sem).wait()
            pltpu.sync_copy(rows, acc.at[dstv], add=True)
            return 0

        lax.fori_loop(0, n_chunks, step, 0)
        plsc.subcore_barrier()
        pltpu.sync_copy(
            acc.at[pl.ds(sid * ROWS_PER_TILE, ROWS_PER_TILE)],
            out_hbm.at[cid, pl.ds(sid * ROWS_PER_TILE, ROWS_PER_TILE)],
        )

    return agg_kernel


def _tc_first(degp_ref, x_ref, w_ref, hs_ref, dinv_ref):
    deg = degp_ref[0, :N] + degp_ref[1, :N] + 1.0
    dinv = lax.rsqrt(deg)[:, None]
    h = jnp.dot(x_ref[...], w_ref[...], preferred_element_type=jnp.float32)
    hs_ref[:N, :] = h * dinv
    hs_ref[N:, :] = jnp.zeros((N_ACC - N, h.shape[1]), jnp.float32)
    dinv_ref[...] = dinv


def _tc_mid(din, dout, p_ref, hs_ref, dinv_ref, b_ref, g_ref, be_ref, w_ref,
            out_ref):
    dinv = dinv_ref[...]
    y = ((p_ref[0, :N, :din] + p_ref[1, :N, :din] + hs_ref[:N, :din]) * dinv
         + b_ref[...])
    mean = jnp.mean(y, axis=0)
    var = jnp.mean((y - mean) ** 2, axis=0)
    yn = (y - mean) * lax.rsqrt(var + EPS) * g_ref[...] + be_ref[...]
    z = jnp.where(yn >= 0, yn, SLOPE * yn)
    h = jnp.dot(z, w_ref[...], preferred_element_type=jnp.float32)
    out_ref[:N, :dout] = h * dinv
    if dout < 128:
        out_ref[:N, dout:] = jnp.zeros((N, 128 - dout), jnp.float32)
    out_ref[N:, :] = jnp.zeros((N_ACC - N, 128), jnp.float32)


def _tc_last(p_ref, hs_ref, dinv_ref, b_ref, out_ref):
    y = (p_ref[0, :N, :] + p_ref[1, :N, :] + hs_ref[:N, :]) * dinv_ref[...]
    out_ref[...] = y + b_ref[...]


def kernel(x, edge_index, W1, b1, g1, be1, W2, b2, g2, be2, W3, b3):
    e = edge_index.shape[1]
    e_pad = ((e + NW * K - 1) // (NW * K)) * (NW * K)
    src = edge_index[0].astype(jnp.int32)
    dst = edge_index[1].astype(jnp.int32)
    pad = jnp.full((e_pad - e,), N, jnp.int32)
    src_p = jnp.concatenate([src, pad])
    dst_p = jnp.concatenate([dst, pad])

    deg_k = _make_deg_kernel(e_pad)
    agg128 = _make_agg_kernel(e_pad, 128)

    degp = deg_k(dst_p)

    hs1, dinv = pl.pallas_call(
        _tc_first,
        out_shape=(
            jax.ShapeDtypeStruct((N_ACC, 128), jnp.float32),
            jax.ShapeDtypeStruct((N, 1), jnp.float32),
        ),
    )(degp, x, W1)

    p1 = agg128(hs1, src_p, dst_p)

    hs2 = pl.pallas_call(
        functools.partial(_tc_mid, 128, 64),
        out_shape=jax.ShapeDtypeStruct((N_ACC, 128), jnp.float32),
    )(p1, hs1, dinv, b1, g1, be1, W2)

    p2 = agg128(hs2, src_p, dst_p)

    hs3 = pl.pallas_call(
        functools.partial(_tc_mid, 64, 128),
        out_shape=jax.ShapeDtypeStruct((N_ACC, 128), jnp.float32),
    )(p2, hs2, dinv, b2, g2, be2, W3)

    p3 = agg128(hs3, src_p, dst_p)

    out = pl.pallas_call(
        _tc_last,
        out_shape=jax.ShapeDtypeStruct((N, 128), jnp.float32),
    )(p3, hs3, dinv, b3)

    return out
